# baseline (speedup 1.0000x reference)
import functools

import jax
import jax.numpy as jnp
from jax import lax
from jax.experimental import pallas as pl
from jax.experimental.pallas import tpu as pltpu
from jax.experimental.pallas import tpu_sc as plsc

B = 2
N = 10000
E = 320000
H = 128

K = 125
TILES = 16
EPT = E // TILES
CHUNKS = EPT // K
ZROWS = 80
ZCH = N // ZROWS
ZITER = -(-ZCH // TILES)
SUPER = 16
NSUPER = CHUNKS // SUPER


def _linear_relu(x, W, b2d, bm):
    M = x.shape[0]

    def body(x_ref, w_ref, b_ref, o_ref):
        o_ref[...] = jnp.maximum(
            jnp.dot(x_ref[...], w_ref[...],
                    preferred_element_type=jnp.float32) + b_ref[...], 0.0)

    return pl.pallas_call(
        body,
        grid=(M // bm,),
        in_specs=[
            pl.BlockSpec((bm, H), lambda i: (i, 0)),
            pl.BlockSpec((H, H), lambda i: (0, 0)),
            pl.BlockSpec((1, H), lambda i: (0, 0)),
        ],
        out_specs=pl.BlockSpec((bm, H), lambda i: (i, 0)),
        out_shape=jax.ShapeDtypeStruct((M, H), jnp.float32),
    )(x, W, b2d)


def _final_head(aggr2, W2, b2d, Wc1, bc1_2d, Wc2, bc2_2d, bm):
    nblocks = (B * N) // bm
    per_batch = N // bm

    def body(a_ref, w2_ref, b2_ref, wc1_ref, bc1_ref, wc2_ref, bc2_ref,
             o_ref, acc_ref):
        i = pl.program_id(0)

        @pl.when(i == 0)
        def _():
            acc_ref[...] = jnp.zeros_like(acc_ref)

        h2 = jnp.maximum(
            jnp.dot(a_ref[...], w2_ref[...],
                    preferred_element_type=jnp.float32) + b2_ref[...], 0.0)
        colsum = jnp.sum(h2, axis=0, keepdims=True)

        @pl.when(i < per_batch)
        def _():
            acc_ref[0:1, :] += colsum

        @pl.when(i >= per_batch)
        def _():
            acc_ref[1:2, :] += colsum

        @pl.when(i == nblocks - 1)
        def _():
            hm = acc_ref[...] / float(N)
            z = jnp.maximum(
                jnp.dot(hm, wc1_ref[...],
                        preferred_element_type=jnp.float32) + bc1_ref[...],
                0.0)
            o_ref[...] = (jnp.dot(z, wc2_ref[...],
                                  preferred_element_type=jnp.float32)
                          + bc2_ref[...])

    return pl.pallas_call(
        body,
        grid=(nblocks,),
        in_specs=[
            pl.BlockSpec((bm, H), lambda i: (i, 0)),
            pl.BlockSpec((H, H), lambda i: (0, 0)),
            pl.BlockSpec((1, H), lambda i: (0, 0)),
            pl.BlockSpec((H, H // 2), lambda i: (0, 0)),
            pl.BlockSpec((1, H // 2), lambda i: (0, 0)),
            pl.BlockSpec((H // 2, 1), lambda i: (0, 0)),
            pl.BlockSpec((1, 1), lambda i: (0, 0)),
        ],
        out_specs=pl.BlockSpec((B, 1), lambda i: (0, 0)),
        out_shape=jax.ShapeDtypeStruct((B, 1), jnp.float32),
        scratch_shapes=[pltpu.VMEM((B, H), jnp.float32)],
    )(aggr2, W2, b2d, Wc1, bc1_2d, Wc2, bc2_2d)


def kernel(x, edge_index, W_embed, b_embed, W1, b1, W2, b2, Wc1, bc1, Wc2, bc2):
    x2 = x.reshape(B * N, H)
    ei = edge_index.astype(jnp.int32)
    rows2d = ei[0].reshape(E // K, K)
    cols2d = ei[1].reshape(E // K, K)
    zeros = jnp.zeros((ZROWS, H), jnp.float32)

    h = _linear_relu(x2, W_embed, b_embed.reshape(1, H), bm=1000)

    aggr1 = _sc_aggregate_2core(h, rows2d, cols2d, zeros)
    h1 = _linear_relu(aggr1, W1, b1.reshape(1, H), bm=1000)
    aggr2 = _sc_aggregate_2core(h1, rows2d, cols2d, zeros)

    out = _final_head(aggr2, W2, b2.reshape(1, H),
                      Wc1, bc1.reshape(1, H // 2),
                      Wc2, bc2.reshape(1, 1), bm=1000)
    return out.reshape(B)


def _sc_aggregate_2core(h, rows2d, cols2d, zeros):
    mesh = plsc.VectorSubcoreMesh(core_axis_name="c", subcore_axis_name="s",
                                  num_cores=2, num_subcores=TILES)

    @functools.partial(
        pl.kernel,
        out_type=jax.ShapeDtypeStruct((B * N, H), jnp.float32),
        mesh=mesh,
        scratch_types=[
            pltpu.VMEM_SHARED((N, H), jnp.float32),
            pltpu.VMEM((SUPER, K), jnp.int32),
            pltpu.VMEM((SUPER, K), jnp.int32),
            pltpu.VMEM((K, H), jnp.float32),
            pltpu.VMEM((K, H), jnp.float32),
            pltpu.SemaphoreType.DMA,
            pltpu.SemaphoreType.DMA,
            pltpu.SemaphoreType.DMA,
            pltpu.SemaphoreType.DMA,
        ],
    )
    def agg(h_hbm, rows_hbm, cols0_hbm, cols1_hbm, zeros_hbm, out_hbm,
            accum, ridx, cidx, buf0, buf1, sg0, sg1, ss0, ss1):
        c = lax.axis_index("c")
        s = lax.axis_index("s")
        zb = buf0.at[pl.ds(0, ZROWS)]

        pltpu.sync_copy(zeros_hbm, zb)
        for z in range(ZITER):
            cid = s + TILES * z

            @pl.when(cid < ZCH)
            def _():
                pltpu.sync_copy(zb, accum.at[pl.ds(cid * ZROWS, ZROWS)])

        plsc.subcore_barrier()

        def super_body(g, _):
            base = s * CHUNKS + g * SUPER
            pltpu.sync_copy(rows_hbm.at[pl.ds(base, SUPER)], ridx)

            @pl.when(c == 0)
            def _():
                pltpu.sync_copy(cols0_hbm.at[pl.ds(base, SUPER)], cidx)

            @pl.when(c == 1)
            def _():
                pltpu.sync_copy(cols1_hbm.at[pl.ds(base, SUPER)], cidx)

            pltpu.async_copy(h_hbm.at[cidx.at[0]], buf0, sg0)

            def pair_body(p, _):
                j0 = 2 * p
                j1 = j0 + 1
                pltpu.async_copy(h_hbm.at[cidx.at[j1]], buf1, sg1)
                pltpu.make_async_copy(h_hbm.at[cidx.at[j0]], buf0, sg0).wait()
                pltpu.async_copy(buf0, accum.at[ridx.at[j0]], ss0, add=True)
                pltpu.make_async_copy(h_hbm.at[cidx.at[j1]], buf1, sg1).wait()
                pltpu.async_copy(buf1, accum.at[ridx.at[j1]], ss1, add=True)
                pltpu.make_async_copy(buf0, accum.at[ridx.at[j0]], ss0).wait()

                @pl.when(j1 + 1 < SUPER)
                def _():
                    pltpu.async_copy(h_hbm.at[cidx.at[j1 + 1]], buf0, sg0)

                pltpu.make_async_copy(buf1, accum.at[ridx.at[j1]], ss1).wait()
                return 0

            lax.fori_loop(0, SUPER // 2, pair_body, 0)
            return 0

        lax.fori_loop(0, NSUPER, super_body, 0)
        plsc.subcore_barrier()

        for z in range(ZITER):
            cid = s + TILES * z

            @pl.when(cid < ZCH)
            def _():
                pltpu.sync_copy(accum.at[pl.ds(cid * ZROWS, ZROWS)], zb)
                pltpu.sync_copy(
                    zb, out_hbm.at[pl.ds(c * N + cid * ZROWS, ZROWS)])

    cols0 = cols2d
    cols1 = cols2d + N
    return agg(h, rows2d, cols0, cols1, zeros)


# speedup vs baseline: 1.4609x; 1.4609x over previous
import functools

import jax
import jax.numpy as jnp
from jax import lax
from jax.experimental import pallas as pl
from jax.experimental.pallas import tpu as pltpu
from jax.experimental.pallas import tpu_sc as plsc

B = 2
N = 10000
E = 320000
H = 128

K = 125
TILES = 16
EPT = E // TILES
CHUNKS = EPT // K
ZROWS = 80
ZCH = N // ZROWS
ZITER = -(-ZCH // TILES)
SUPER = 16
NSUPER = CHUNKS // SUPER


def _linear_relu(x, W, b2d, bm):
    M = x.shape[0]

    def body(x_ref, w_ref, b_ref, o_ref):
        acc = jnp.maximum(
            jnp.dot(x_ref[...], w_ref[...],
                    preferred_element_type=jnp.float32) + b_ref[...], 0.0)
        o_ref[...] = acc.astype(jnp.bfloat16)

    return pl.pallas_call(
        body,
        grid=(M // bm,),
        in_specs=[
            pl.BlockSpec((bm, H), lambda i: (i, 0)),
            pl.BlockSpec((H, H), lambda i: (0, 0)),
            pl.BlockSpec((1, H), lambda i: (0, 0)),
        ],
        out_specs=pl.BlockSpec((bm, H), lambda i: (i, 0)),
        out_shape=jax.ShapeDtypeStruct((M, H), jnp.bfloat16),
    )(x, W, b2d)


def _final_head(aggr2, W2, b2d, Wc1, bc1_2d, Wc2, bc2_2d, bm):
    nblocks = (B * N) // bm
    per_batch = N // bm

    def body(a_ref, w2_ref, b2_ref, wc1_ref, bc1_ref, wc2_ref, bc2_ref,
             o_ref, acc_ref):
        i = pl.program_id(0)

        @pl.when(i == 0)
        def _():
            acc_ref[...] = jnp.zeros_like(acc_ref)

        h2 = jnp.maximum(
            jnp.dot(a_ref[...], w2_ref[...],
                    preferred_element_type=jnp.float32) + b2_ref[...], 0.0)
        colsum = jnp.sum(h2, axis=0, keepdims=True)

        @pl.when(i < per_batch)
        def _():
            acc_ref[0:1, :] += colsum

        @pl.when(i >= per_batch)
        def _():
            acc_ref[1:2, :] += colsum

        @pl.when(i == nblocks - 1)
        def _():
            hm = acc_ref[...] / float(N)
            z = jnp.maximum(
                jnp.dot(hm, wc1_ref[...],
                        preferred_element_type=jnp.float32) + bc1_ref[...],
                0.0)
            o_ref[...] = (jnp.dot(z, wc2_ref[...],
                                  preferred_element_type=jnp.float32)
                          + bc2_ref[...])

    return pl.pallas_call(
        body,
        grid=(nblocks,),
        in_specs=[
            pl.BlockSpec((bm, H), lambda i: (i, 0)),
            pl.BlockSpec((H, H), lambda i: (0, 0)),
            pl.BlockSpec((1, H), lambda i: (0, 0)),
            pl.BlockSpec((H, H // 2), lambda i: (0, 0)),
            pl.BlockSpec((1, H // 2), lambda i: (0, 0)),
            pl.BlockSpec((H // 2, 1), lambda i: (0, 0)),
            pl.BlockSpec((1, 1), lambda i: (0, 0)),
        ],
        out_specs=pl.BlockSpec((B, 1), lambda i: (0, 0)),
        out_shape=jax.ShapeDtypeStruct((B, 1), jnp.float32),
        scratch_shapes=[pltpu.VMEM((B, H), jnp.float32)],
    )(aggr2, W2, b2d, Wc1, bc1_2d, Wc2, bc2_2d)


def kernel(x, edge_index, W_embed, b_embed, W1, b1, W2, b2, Wc1, bc1, Wc2, bc2):
    x2 = x.reshape(B * N, H)
    ei = edge_index.astype(jnp.int32)
    rows2d = ei[0].reshape(E // K, K)
    cols2d = ei[1].reshape(E // K, K)
    zeros = jnp.zeros((ZROWS, H), jnp.bfloat16)

    h = _linear_relu(x2, W_embed, b_embed.reshape(1, H), bm=1000)

    aggr1 = _sc_aggregate_2core(h, rows2d, cols2d, zeros)
    h1 = _linear_relu(aggr1, W1, b1.reshape(1, H), bm=1000)
    aggr2 = _sc_aggregate_2core(h1, rows2d, cols2d, zeros)

    out = _final_head(aggr2, W2, b2.reshape(1, H),
                      Wc1, bc1.reshape(1, H // 2),
                      Wc2, bc2.reshape(1, 1), bm=1000)
    return out.reshape(B)


def _sc_aggregate_2core(h, rows2d, cols2d, zeros):
    mesh = plsc.VectorSubcoreMesh(core_axis_name="c", subcore_axis_name="s",
                                  num_cores=2, num_subcores=TILES)

    @functools.partial(
        pl.kernel,
        out_type=jax.ShapeDtypeStruct((B * N, H), jnp.bfloat16),
        mesh=mesh,
        scratch_types=[
            pltpu.VMEM_SHARED((N, H), jnp.bfloat16),
            pltpu.VMEM((SUPER, K), jnp.int32),
            pltpu.VMEM((SUPER, K), jnp.int32),
            pltpu.VMEM((K, H), jnp.bfloat16),
            pltpu.VMEM((K, H), jnp.bfloat16),
            pltpu.SemaphoreType.DMA,
            pltpu.SemaphoreType.DMA,
        ],
        compiler_params=pltpu.CompilerParams(use_tc_tiling_on_sc=False),
    )
    def agg(h_hbm, rows_hbm, cols0_hbm, cols1_hbm, zeros_hbm, out_hbm,
            accum, ridx, cidx, buf0, buf1, sg0, sg1):
        c = lax.axis_index("c")
        s = lax.axis_index("s")
        zb = buf0.at[pl.ds(0, ZROWS)]

        pltpu.sync_copy(zeros_hbm, zb)
        for z in range(ZITER):
            cid = s + TILES * z

            @pl.when(cid < ZCH)
            def _():
                pltpu.sync_copy(zb, accum.at[pl.ds(cid * ZROWS, ZROWS)])

        plsc.subcore_barrier()

        def super_body(g, _):
            base = s * CHUNKS + g * SUPER
            pltpu.sync_copy(rows_hbm.at[pl.ds(base, SUPER)], ridx)

            @pl.when(c == 0)
            def _():
                pltpu.sync_copy(cols0_hbm.at[pl.ds(base, SUPER)], cidx)

            @pl.when(c == 1)
            def _():
                pltpu.sync_copy(cols1_hbm.at[pl.ds(base, SUPER)], cidx)

            pltpu.async_copy(h_hbm.at[cidx.at[0]], buf0, sg0)

            def pair_body(p, _):
                j0 = 2 * p
                j1 = j0 + 1
                pltpu.async_copy(h_hbm.at[cidx.at[j1]], buf1, sg1)
                pltpu.make_async_copy(h_hbm.at[cidx.at[j0]], buf0, sg0).wait()
                pltpu.sync_copy(buf0, accum.at[ridx.at[j0]], add=True)

                @pl.when(j1 + 1 < SUPER)
                def _():
                    pltpu.async_copy(h_hbm.at[cidx.at[j1 + 1]], buf0, sg0)

                pltpu.make_async_copy(h_hbm.at[cidx.at[j1]], buf1, sg1).wait()
                pltpu.sync_copy(buf1, accum.at[ridx.at[j1]], add=True)
                return 0

            lax.fori_loop(0, SUPER // 2, pair_body, 0)
            return 0

        lax.fori_loop(0, NSUPER, super_body, 0)
        plsc.subcore_barrier()

        for z in range(ZITER):
            cid = s + TILES * z

            @pl.when(cid < ZCH)
            def _():
                pltpu.sync_copy(accum.at[pl.ds(cid * ZROWS, ZROWS)], zb)
                pltpu.sync_copy(
                    zb, out_hbm.at[pl.ds(c * N + cid * ZROWS, ZROWS)])

    cols0 = cols2d
    cols1 = cols2d + N
    return agg(h, rows2d, cols0, cols1, zeros)


# speedup vs baseline: 1.5024x; 1.0284x over previous
import functools

import jax
import jax.numpy as jnp
from jax import lax
from jax.experimental import pallas as pl
from jax.experimental.pallas import tpu as pltpu
from jax.experimental.pallas import tpu_sc as plsc

B = 2
N = 10000
E = 320000
H = 128

K = 125
TILES = 16
EPT = E // TILES
CHUNKS = EPT // K
ZROWS = 80
ZCH = N // ZROWS
ZITER = -(-ZCH // TILES)
SUPER = 16
NSUPER = CHUNKS // SUPER


def _linear_relu(x, W, b2d, bm):
    M = x.shape[0]
    half = (M // bm) // 2

    def body(x_ref, w_ref, b_ref, o0_ref, o1_ref):
        i = pl.program_id(0)
        acc = jnp.maximum(
            jnp.dot(x_ref[...], w_ref[...],
                    preferred_element_type=jnp.float32) + b_ref[...], 0.0)

        @pl.when(i < half)
        def _():
            o0_ref[...] = acc.astype(jnp.bfloat16)

        @pl.when(i >= half)
        def _():
            o1_ref[...] = acc.astype(jnp.bfloat16)

    return pl.pallas_call(
        body,
        grid=(M // bm,),
        in_specs=[
            pl.BlockSpec((bm, H), lambda i: (i, 0)),
            pl.BlockSpec((H, H), lambda i: (0, 0)),
            pl.BlockSpec((1, H), lambda i: (0, 0)),
        ],
        out_specs=[
            pl.BlockSpec((bm, H), lambda i: (jnp.minimum(i, half - 1), 0)),
            pl.BlockSpec((bm, H), lambda i: (jnp.maximum(i - half, 0), 0)),
        ],
        out_shape=[jax.ShapeDtypeStruct((N, H), jnp.bfloat16),
                   jax.ShapeDtypeStruct((N, H), jnp.bfloat16)],
    )(x, W, b2d)


def _final_head(aggr2, W2, b2d, Wc1, bc1_2d, Wc2, bc2_2d, bm):
    nblocks = (B * N) // bm
    per_batch = N // bm

    def body(a_ref, w2_ref, b2_ref, wc1_ref, bc1_ref, wc2_ref, bc2_ref,
             o_ref, acc_ref):
        i = pl.program_id(0)

        @pl.when(i == 0)
        def _():
            acc_ref[...] = jnp.zeros_like(acc_ref)

        h2 = jnp.maximum(
            jnp.dot(a_ref[...], w2_ref[...],
                    preferred_element_type=jnp.float32) + b2_ref[...], 0.0)
        colsum = jnp.sum(h2, axis=0, keepdims=True)

        @pl.when(i < per_batch)
        def _():
            acc_ref[0:1, :] += colsum

        @pl.when(i >= per_batch)
        def _():
            acc_ref[1:2, :] += colsum

        @pl.when(i == nblocks - 1)
        def _():
            hm = acc_ref[...] / float(N)
            z = jnp.maximum(
                jnp.dot(hm, wc1_ref[...],
                        preferred_element_type=jnp.float32) + bc1_ref[...],
                0.0)
            o_ref[...] = (jnp.dot(z, wc2_ref[...],
                                  preferred_element_type=jnp.float32)
                          + bc2_ref[...])

    return pl.pallas_call(
        body,
        grid=(nblocks,),
        in_specs=[
            pl.BlockSpec((bm, H), lambda i: (i, 0)),
            pl.BlockSpec((H, H), lambda i: (0, 0)),
            pl.BlockSpec((1, H), lambda i: (0, 0)),
            pl.BlockSpec((H, H // 2), lambda i: (0, 0)),
            pl.BlockSpec((1, H // 2), lambda i: (0, 0)),
            pl.BlockSpec((H // 2, 1), lambda i: (0, 0)),
            pl.BlockSpec((1, 1), lambda i: (0, 0)),
        ],
        out_specs=pl.BlockSpec((B, 1), lambda i: (0, 0)),
        out_shape=jax.ShapeDtypeStruct((B, 1), jnp.float32),
        scratch_shapes=[pltpu.VMEM((B, H), jnp.float32)],
    )(aggr2, W2, b2d, Wc1, bc1_2d, Wc2, bc2_2d)


def kernel(x, edge_index, W_embed, b_embed, W1, b1, W2, b2, Wc1, bc1, Wc2, bc2):
    x2 = x.reshape(B * N, H)
    rc = edge_index.astype(jnp.int32).reshape(2, E // K, K)
    zeros = jnp.zeros((ZROWS, H), jnp.bfloat16)

    h0, h1 = _linear_relu(x2, W_embed, b_embed.reshape(1, H), bm=1000)

    aggr1 = _sc_aggregate_2core(h0, h1, rc, zeros)
    g0, g1 = _linear_relu(aggr1, W1, b1.reshape(1, H), bm=1000)
    aggr2 = _sc_aggregate_2core(g0, g1, rc, zeros)

    out = _final_head(aggr2, W2, b2.reshape(1, H),
                      Wc1, bc1.reshape(1, H // 2),
                      Wc2, bc2.reshape(1, 1), bm=1000)
    return out.reshape(B)


def _sc_aggregate_2core(h0, h1, rc, zeros):
    mesh = plsc.VectorSubcoreMesh(core_axis_name="c", subcore_axis_name="s",
                                  num_cores=2, num_subcores=TILES)

    @functools.partial(
        pl.kernel,
        out_type=jax.ShapeDtypeStruct((B * N, H), jnp.bfloat16),
        mesh=mesh,
        scratch_types=[
            pltpu.VMEM_SHARED((N, H), jnp.bfloat16),
            pltpu.VMEM((SUPER, K), jnp.int32),
            pltpu.VMEM((SUPER, K), jnp.int32),
            pltpu.VMEM((K, H), jnp.bfloat16),
            pltpu.VMEM((K, H), jnp.bfloat16),
            pltpu.SemaphoreType.DMA,
            pltpu.SemaphoreType.DMA,
        ],
        compiler_params=pltpu.CompilerParams(use_tc_tiling_on_sc=False),
    )
    def agg(h0_hbm, h1_hbm, rc_hbm, zeros_hbm, out_hbm,
            accum, ridx, cidx, buf0, buf1, sg0, sg1):
        c = lax.axis_index("c")
        s = lax.axis_index("s")
        zb = buf0.at[pl.ds(0, ZROWS)]

        pltpu.sync_copy(zeros_hbm, zb)
        for z in range(ZITER):
            cid = s + TILES * z

            @pl.when(cid < ZCH)
            def _():
                pltpu.sync_copy(zb, accum.at[pl.ds(cid * ZROWS, ZROWS)])

        plsc.subcore_barrier()

        def make_super_body(h_hbm):
            def super_body(g, _):
                base = s * CHUNKS + g * SUPER
                pltpu.sync_copy(rc_hbm.at[0].at[pl.ds(base, SUPER)], ridx)
                pltpu.sync_copy(rc_hbm.at[1].at[pl.ds(base, SUPER)], cidx)

                pltpu.async_copy(h_hbm.at[cidx.at[0]], buf0, sg0)

                def pair_body(p, _):
                    j0 = 2 * p
                    j1 = j0 + 1
                    pltpu.async_copy(h_hbm.at[cidx.at[j1]], buf1, sg1)
                    pltpu.make_async_copy(
                        h_hbm.at[cidx.at[j0]], buf0, sg0).wait()
                    pltpu.sync_copy(buf0, accum.at[ridx.at[j0]], add=True)

                    @pl.when(j1 + 1 < SUPER)
                    def _():
                        pltpu.async_copy(h_hbm.at[cidx.at[j1 + 1]], buf0, sg0)

                    pltpu.make_async_copy(
                        h_hbm.at[cidx.at[j1]], buf1, sg1).wait()
                    pltpu.sync_copy(buf1, accum.at[ridx.at[j1]], add=True)
                    return 0

                lax.fori_loop(0, SUPER // 2, pair_body, 0)
                return 0

            return super_body

        @pl.when(c == 0)
        def _():
            lax.fori_loop(0, NSUPER, make_super_body(h0_hbm), 0)

        @pl.when(c == 1)
        def _():
            lax.fori_loop(0, NSUPER, make_super_body(h1_hbm), 0)
        plsc.subcore_barrier()

        for z in range(ZITER):
            cid = s + TILES * z

            @pl.when(cid < ZCH)
            def _():
                pltpu.sync_copy(accum.at[pl.ds(cid * ZROWS, ZROWS)], zb)
                pltpu.sync_copy(
                    zb, out_hbm.at[pl.ds(c * N + cid * ZROWS, ZROWS)])

    return agg(h0, h1, rc, zeros)
